# Initial kernel scaffold; baseline (speedup 1.0000x reference)
import jax
import jax.numpy as jnp
from jax.experimental import pallas as pl


def kernel(x, W):
    raise NotImplementedError("write your pallas kernel here")


# speedup vs baseline: 1.3338x; 1.3338x over previous
import functools

import jax
import jax.numpy as jnp
from jax import lax
from jax.experimental import pallas as pl
from jax.experimental.pallas import tpu as pltpu
from jax.experimental.pallas import tpu_sc as plsc

N_E = 8192
E_DIM = 32
N_TOK = 16384
HALF = N_E // 2

TOK_BLK = 256

_SC_CORES = 2
_SC_SUBCORES = 16
_NW = _SC_CORES * _SC_SUBCORES
_ROWS_PER_W = N_TOK // _NW


def _argmin_body(x_ref, wt_ref, xsq_ref, wsq_ref, idx_ref):
    xb = x_ref[...]
    wt = wt_ref[...]
    m = jnp.dot(xb, wt, preferred_element_type=jnp.float32)
    d = (xsq_ref[...] + wsq_ref[...]) - 2.0 * m
    dA = d[:, :HALF]
    dB = d[:, HALF:]
    vA = jnp.min(dA, axis=1, keepdims=True)
    vB = jnp.min(dB, axis=1, keepdims=True)
    cols = lax.broadcasted_iota(jnp.int32, (TOK_BLK, HALF), 1)
    iA = jnp.min(jnp.where(dA == vA, cols, N_E), axis=1, keepdims=True)
    iB = jnp.min(jnp.where(dB == vB, cols, N_E), axis=1, keepdims=True) + HALF
    vab = vA.astype(jnp.bfloat16).astype(jnp.float32)
    idx_ref[...] = jnp.where(vab <= vB, iA, iB)


def _compute_indices(x, wt, xsq, wsq):
    grid = N_TOK // TOK_BLK
    return pl.pallas_call(
        _argmin_body,
        grid=(grid,),
        in_specs=[
            pl.BlockSpec((TOK_BLK, E_DIM), lambda i: (i, 0)),
            pl.BlockSpec((E_DIM, N_E), lambda i: (0, 0)),
            pl.BlockSpec((TOK_BLK, 1), lambda i: (i, 0)),
            pl.BlockSpec((1, N_E), lambda i: (0, 0)),
        ],
        out_specs=pl.BlockSpec((TOK_BLK, 1), lambda i: (i, 0)),
        out_shape=jax.ShapeDtypeStruct((N_TOK, 1), jnp.int32),
    )(x, wt, xsq, wsq).reshape(N_TOK)


_CHUNK = 128
_CHUNKS_PER_W = _ROWS_PER_W // _CHUNK


@functools.lru_cache(maxsize=1)
def _make_sc_gather():
    @functools.partial(
        pl.kernel,
        mesh=plsc.VectorSubcoreMesh(core_axis_name="c", subcore_axis_name="s"),
        out_type=jax.ShapeDtypeStruct((N_TOK, E_DIM), jnp.float32),
        scratch_types=[
            pltpu.VMEM((_CHUNKS_PER_W, _CHUNK), jnp.int32),
            pltpu.VMEM((_ROWS_PER_W, E_DIM), jnp.float32),
            pltpu.SemaphoreType.DMA,
        ],
        compiler_params=pltpu.CompilerParams(use_tc_tiling_on_sc=False),
    )
    def _sc_gather(table_hbm, idx_hbm, out_hbm, idx_v, rows_v, sem):
        wid = lax.axis_index("s") * _SC_CORES + lax.axis_index("c")
        pltpu.sync_copy(idx_hbm.at[pl.ds(wid * _CHUNKS_PER_W, _CHUNKS_PER_W)], idx_v)
        copies = [
            pltpu.async_copy(
                table_hbm.at[idx_v.at[j]],
                rows_v.at[pl.ds(j * _CHUNK, _CHUNK)],
                sem,
            )
            for j in range(_CHUNKS_PER_W)
        ]
        for c in copies:
            c.wait()
        pltpu.sync_copy(rows_v, out_hbm.at[pl.ds(wid * _ROWS_PER_W, _ROWS_PER_W)])

    return _sc_gather


def kernel(x, W):
    xsq = jnp.sum(x ** 2, axis=1).reshape(N_TOK, 1)
    wsq = jnp.sum(W ** 2, axis=1).reshape(1, N_E)
    idx = _compute_indices(x, W.T, xsq, wsq)
    z_q = _make_sc_gather()(W, idx.reshape(N_TOK // _CHUNK, _CHUNK))
    return (z_q, idx)
